# baseline (speedup 1.0000x reference)
import functools

import jax
import jax.numpy as jnp
from jax import lax
from jax.experimental import pallas as pl
from jax.experimental.pallas import tpu as pltpu
from jax.experimental.pallas import tpu_sc as plsc

N = 10000
E = 320000
CHUNK = 128
NC, NS = 2, 16
NW = NC * NS
SUP = 40
GROUPS = 2
CPW = SUP * GROUPS
NCHUNK = NW * CPW
E_PAD = NCHUNK * CHUNK
ACC_ROWS = 10240
RPT = ACC_ROWS // NS

_mesh = plsc.VectorSubcoreMesh(core_axis_name="c", subcore_axis_name="s")


def _deg_body(dst_hbm, out_hbm, didx, rows, acc, ssem):
    c = lax.axis_index("c")
    s = lax.axis_index("s")
    zrow = jnp.zeros((16,), jnp.float32)
    onerow = jnp.full((16,), 1.0, jnp.float32)

    def zfill(i, _):
        for j in range(8):
            rows[i, pl.ds(j * 16, 16)] = zrow
        return 0

    lax.fori_loop(0, CHUNK, zfill, 0)
    for k in range(RPT // CHUNK):
        pltpu.sync_copy(rows, acc.at[pl.ds(s * RPT + k * CHUNK, CHUNK)])

    def ofill(i, _):
        for j in range(8):
            rows[i, pl.ds(j * 16, 16)] = onerow
        return 0

    lax.fori_loop(0, CHUNK, ofill, 0)
    plsc.subcore_barrier()

    base = (c * NS + s) * CPW
    pltpu.sync_copy(dst_hbm.at[pl.ds(base, CPW)], didx)

    LAG = 8

    def step(i, _):
        pltpu.async_copy(rows, acc.at[didx.at[i]], ssem, add=True)

        @pl.when(i >= LAG)
        def _():
            pltpu.make_async_copy(rows, acc.at[didx.at[i - LAG]], ssem).wait()

        return 0

    lax.fori_loop(0, CPW, step, 0)

    def drain(i, _):
        pltpu.make_async_copy(rows, acc.at[didx.at[CPW - LAG + i]], ssem).wait()
        return 0

    lax.fori_loop(0, LAG, drain, 0)
    plsc.subcore_barrier()
    r0 = s * RPT
    pltpu.sync_copy(acc.at[pl.ds(r0, RPT)], out_hbm.at[c, pl.ds(r0, RPT)])


_deg_kernel = functools.partial(
    pl.kernel,
    out_type=jax.ShapeDtypeStruct((NC, ACC_ROWS, 128), jnp.float32),
    mesh=_mesh,
    scratch_types=[
        pltpu.VMEM((CPW, CHUNK), jnp.int32),
        pltpu.VMEM((CHUNK, 128), jnp.float32),
        pltpu.VMEM_SHARED((ACC_ROWS, 128), jnp.float32),
        pltpu.SemaphoreType.DMA,
    ],
)(_deg_body)


def _agg_body(src_hbm, dst_hbm, g_hbm, out_hbm, sidxg, didxg, rows0, rows1,
              acc, gsem0, gsem1, ssem0, ssem1):
    c = lax.axis_index("c")
    s = lax.axis_index("s")
    zrow = jnp.zeros((16,), jnp.float32)

    def zfill(i, _):
        for j in range(8):
            rows0[i, pl.ds(j * 16, 16)] = zrow
        return 0

    lax.fori_loop(0, CHUNK, zfill, 0)
    for k in range(RPT // CHUNK):
        pltpu.sync_copy(rows0, acc.at[pl.ds(s * RPT + k * CHUNK, CHUNK)])
    plsc.subcore_barrier()

    base = (c * NS + s) * CPW

    for g in range(GROUPS):
        off = base + g * SUP
        pltpu.sync_copy(src_hbm.at[pl.ds(off, SUP)], sidxg)
        pltpu.sync_copy(dst_hbm.at[pl.ds(off, SUP)], didxg)
        pltpu.async_copy(g_hbm.at[sidxg.at[0]], rows0, gsem0)
        pltpu.async_copy(g_hbm.at[sidxg.at[1]], rows1, gsem1)

        def pair(k, _):
            j0 = 2 * k
            j1 = 2 * k + 1
            pltpu.make_async_copy(g_hbm.at[sidxg.at[j0]], rows0, gsem0).wait()
            pltpu.async_copy(rows0, acc.at[didxg.at[j0]], ssem0, add=True)
            pltpu.make_async_copy(g_hbm.at[sidxg.at[j1]], rows1, gsem1).wait()
            pltpu.async_copy(rows1, acc.at[didxg.at[j1]], ssem1, add=True)

            @pl.when(k < SUP // 2 - 1)
            def _():
                pltpu.make_async_copy(rows0, acc.at[didxg.at[j0]], ssem0).wait()
                pltpu.async_copy(g_hbm.at[sidxg.at[j0 + 2]], rows0, gsem0)
                pltpu.make_async_copy(rows1, acc.at[didxg.at[j1]], ssem1).wait()
                pltpu.async_copy(g_hbm.at[sidxg.at[j1 + 2]], rows1, gsem1)

            return 0

        lax.fori_loop(0, SUP // 2, pair, 0)
        pltpu.make_async_copy(rows0, acc.at[didxg.at[SUP - 2]], ssem0).wait()
        pltpu.make_async_copy(rows1, acc.at[didxg.at[SUP - 1]], ssem1).wait()

    plsc.subcore_barrier()
    r0 = s * RPT
    pltpu.sync_copy(acc.at[pl.ds(r0, RPT)], out_hbm.at[c, pl.ds(r0, RPT)])


_agg_kernel = functools.partial(
    pl.kernel,
    out_type=jax.ShapeDtypeStruct((NC, ACC_ROWS, 128), jnp.float32),
    mesh=_mesh,
    scratch_types=[
        pltpu.VMEM((SUP, CHUNK), jnp.int32),
        pltpu.VMEM((SUP, CHUNK), jnp.int32),
        pltpu.VMEM((CHUNK, 128), jnp.float32),
        pltpu.VMEM((CHUNK, 128), jnp.float32),
        pltpu.VMEM_SHARED((ACC_ROWS, 128), jnp.float32),
        pltpu.SemaphoreType.DMA,
        pltpu.SemaphoreType.DMA,
        pltpu.SemaphoreType.DMA,
        pltpu.SemaphoreType.DMA,
    ],
)(_agg_body)


_BR = 1000


def _dis_block(d0r, d1r):
    deg = d0r[...] + d1r[...] + 1.0
    return lax.rsqrt(deg)


def _scale_mm_body(xr, wr, d0r, d1r, gr):
    h = jnp.dot(xr[...], wr[...], preferred_element_type=jnp.float32)
    gr[...] = h * _dis_block(d0r, d1r)


def _layer2_body(a0r, a1r, gr, d0r, d1r, br, wr, or_):
    dis = _dis_block(d0r, d1r)
    h = jnp.tanh(dis * (a0r[...] + a1r[...] + gr[...]) + br[...])
    or_[...] = jnp.dot(h, wr[...], preferred_element_type=jnp.float32) * dis


def _final_body(a0r, a1r, gr, d0r, d1r, br, wcr, bcr, outr, hr):
    dis = _dis_block(d0r, d1r)
    h = jnp.tanh(dis * (a0r[...] + a1r[...] + gr[...]) + br[...])
    hr[...] = h
    outr[...] = jnp.dot(h, wcr[...], preferred_element_type=jnp.float32) + bcr[...]


def _rows(bs):
    return pl.BlockSpec((_BR, bs), lambda i: (i, 0))


def _full(shape):
    return pl.BlockSpec(shape, lambda i: tuple(0 for _ in shape))


def kernel(x, edge_index, W1, b1, W2, b2, Wc, bc):
    src = edge_index[0]
    dst = edge_index[1]
    pad = E_PAD - E
    src_p = jnp.concatenate([src, jnp.zeros((pad,), jnp.int32)]).reshape(NCHUNK, CHUNK)
    dst_p = jnp.concatenate([dst, jnp.full((pad,), N, jnp.int32)]).reshape(NCHUNK, CHUNK)
    b1r = b1.reshape(1, 128)
    b2r = b2.reshape(1, 128)
    bcr = bc.reshape(1, 64)

    deg_parts = _deg_kernel(dst_p)
    d0, d1 = deg_parts[0], deg_parts[1]

    grid = N // _BR
    g1 = pl.pallas_call(
        _scale_mm_body,
        grid=(grid,),
        in_specs=[_rows(128), _full((128, 128)), _rows(128), _rows(128)],
        out_specs=_rows(128),
        out_shape=jax.ShapeDtypeStruct((N, 128), jnp.float32),
    )(x, W1, d0, d1)

    agg1 = _agg_kernel(src_p, dst_p, g1)

    g2 = pl.pallas_call(
        _layer2_body,
        grid=(grid,),
        in_specs=[_rows(128), _rows(128), _rows(128), _rows(128), _rows(128),
                  _full((1, 128)), _full((128, 128))],
        out_specs=_rows(128),
        out_shape=jax.ShapeDtypeStruct((N, 128), jnp.float32),
    )(agg1[0], agg1[1], g1, d0, d1, b1r, W2)

    agg2 = _agg_kernel(src_p, dst_p, g2)

    out, h2 = pl.pallas_call(
        _final_body,
        grid=(grid,),
        in_specs=[_rows(128), _rows(128), _rows(128), _rows(128), _rows(128),
                  _full((1, 128)), _full((128, 64)), _full((1, 64))],
        out_specs=[_rows(64), _rows(128)],
        out_shape=[jax.ShapeDtypeStruct((N, 64), jnp.float32),
                   jax.ShapeDtypeStruct((N, 128), jnp.float32)],
    )(agg2[0], agg2[1], g2, d0, d1, b2r, Wc, bcr)

    return (out, h2)
